# baseline (speedup 1.0000x reference)
import jax
import jax.numpy as jnp
from jax import lax
from jax.experimental import pallas as pl
from jax.experimental.pallas import tpu as pltpu, tpu_sc as plsc

_NC, _NS = 2, 16
_NW = _NC * _NS
_B = 16384
_DIM = 64
_PAD = 128
_OUTW = 3 * _DIM
_BPW = _B // _NW
_CHUNK = 64
_NR = _BPW // _CHUNK
_LANES = 16


def _compact_round(hbuf, rbuf, tbuf, obuf):
    for row in range(_CHUNK):
        out_base = row * _OUTW
        for t, buf in enumerate((hbuf, rbuf, tbuf)):
            for g in range(_DIM // _LANES):
                v = buf[row, pl.ds(g * _LANES, _LANES)]
                obuf[pl.ds(out_base + t * _DIM + g * _LANES, _LANES)] = v


def _gather_body(ent_hbm, rel_hbm, h_hbm, r_hbm, t_hbm, out_hbm,
                 hidx, ridx, tidx, hbuf, rbuf, tbuf, obuf, sem):
    wid = lax.axis_index("s") * _NC + lax.axis_index("c")
    base = wid * _BPW
    pltpu.sync_copy(h_hbm.at[pl.ds(base, _BPW)], hidx)
    pltpu.sync_copy(r_hbm.at[pl.ds(base, _BPW)], ridx)
    pltpu.sync_copy(t_hbm.at[pl.ds(base, _BPW)], tidx)

    def round_body(j):
        s = pl.ds(j * _CHUNK, _CHUNK)
        ch = pltpu.async_copy(ent_hbm.at[hidx.at[s]], hbuf, sem)
        cr = pltpu.async_copy(rel_hbm.at[ridx.at[s]], rbuf, sem)
        ct = pltpu.async_copy(ent_hbm.at[tidx.at[s]], tbuf, sem)
        ch.wait()
        cr.wait()
        ct.wait()
        _compact_round(hbuf, rbuf, tbuf, obuf)
        pltpu.sync_copy(
            obuf,
            out_hbm.at[pl.ds((base + j * _CHUNK) * _OUTW, _CHUNK * _OUTW)])

    lax.fori_loop(0, _NR, lambda j, _: (round_body(j), None)[1], None)


def _pad_body(x_ref, o_ref):
    o_ref[...] = jnp.concatenate(
        [x_ref[...].T, jnp.zeros((_PAD, _PAD - _DIM), jnp.float32)], axis=1)


def _pad_table_tc(table):
    n = table.shape[0]
    grid = (n + _PAD - 1) // _PAD
    return pl.pallas_call(
        _pad_body,
        grid=(grid,),
        in_specs=[pl.BlockSpec((_DIM, _PAD), lambda i: (0, i))],
        out_specs=pl.BlockSpec((_PAD, _PAD), lambda i: (i, 0)),
        out_shape=jax.ShapeDtypeStruct((n, _PAD), jnp.float32),
    )(table.T)


def kernel(entity_table, relation_table, h_idx, r_idx, t_idx, y):
    ent_pad = _pad_table_tc(entity_table)
    rel_pad = _pad_table_tc(relation_table)
    mesh = plsc.VectorSubcoreMesh(core_axis_name="c", subcore_axis_name="s")
    out = pl.kernel(
        _gather_body,
        out_type=jax.ShapeDtypeStruct((_B * _OUTW,), jnp.float32),
        mesh=mesh,
        compiler_params=pltpu.CompilerParams(use_tc_tiling_on_sc=True),
        scratch_types=[
            pltpu.VMEM((_BPW,), jnp.int32),
            pltpu.VMEM((_BPW,), jnp.int32),
            pltpu.VMEM((_BPW,), jnp.int32),
            pltpu.VMEM((_CHUNK, _PAD), jnp.float32),
            pltpu.VMEM((_CHUNK, _PAD), jnp.float32),
            pltpu.VMEM((_CHUNK, _PAD), jnp.float32),
            pltpu.VMEM((_CHUNK * _OUTW,), jnp.float32),
            pltpu.SemaphoreType.DMA,
        ],
    )(ent_pad, rel_pad,
      h_idx.astype(jnp.int32), r_idx.astype(jnp.int32), t_idx.astype(jnp.int32))
    return out


# speedup vs baseline: 4.8508x; 4.8508x over previous
import jax
import jax.numpy as jnp
from jax import lax
from jax.experimental import pallas as pl
from jax.experimental.pallas import tpu as pltpu, tpu_sc as plsc

_NC, _NS = 2, 16
_NW = _NC * _NS
_B = 16384
_DIM = 64
_OUTW = 3 * _DIM
_BPW = _B // _NW
_STREAM = 128
_NCHUNK = _BPW // _STREAM


def _gather_body(ent_hbm, rel_hbm, h_hbm, r_hbm, t_hbm, out_hbm,
                 hidx, ridx, tidx, hbuf, rbuf, tbuf, sem):
    wid = lax.axis_index("s") * _NC + lax.axis_index("c")
    base = wid * _BPW
    pltpu.sync_copy(h_hbm.at[pl.ds(base, _BPW)], hidx)
    pltpu.sync_copy(r_hbm.at[pl.ds(base, _BPW)], ridx)
    pltpu.sync_copy(t_hbm.at[pl.ds(base, _BPW)], tidx)

    copies = []
    for c in range(_NCHUNK):
        s = pl.ds(c * _STREAM, _STREAM)
        copies.append(pltpu.async_copy(ent_hbm.at[hidx.at[s]], hbuf.at[s], sem))
        copies.append(pltpu.async_copy(rel_hbm.at[ridx.at[s]], rbuf.at[s], sem))
        copies.append(pltpu.async_copy(ent_hbm.at[tidx.at[s]], tbuf.at[s], sem))
    for cp in copies:
        cp.wait()

    rows = pl.ds(base, _BPW)
    pltpu.sync_copy(hbuf, out_hbm.at[rows, pl.ds(0, _DIM)])
    pltpu.sync_copy(rbuf, out_hbm.at[rows, pl.ds(_DIM, _DIM)])
    pltpu.sync_copy(tbuf, out_hbm.at[rows, pl.ds(2 * _DIM, _DIM)])


def kernel(entity_table, relation_table, h_idx, r_idx, t_idx, y):
    mesh = plsc.VectorSubcoreMesh(core_axis_name="c", subcore_axis_name="s")
    out = pl.kernel(
        _gather_body,
        out_type=jax.ShapeDtypeStruct((_B, _OUTW), jnp.float32),
        mesh=mesh,
        compiler_params=pltpu.CompilerParams(use_tc_tiling_on_sc=False),
        scratch_types=[
            pltpu.VMEM((_BPW,), jnp.int32),
            pltpu.VMEM((_BPW,), jnp.int32),
            pltpu.VMEM((_BPW,), jnp.int32),
            pltpu.VMEM((_BPW, _DIM), jnp.float32),
            pltpu.VMEM((_BPW, _DIM), jnp.float32),
            pltpu.VMEM((_BPW, _DIM), jnp.float32),
            pltpu.SemaphoreType.DMA,
        ],
    )(entity_table, relation_table,
      h_idx.astype(jnp.int32), r_idx.astype(jnp.int32), t_idx.astype(jnp.int32))
    return out.reshape(-1)


# speedup vs baseline: 5.0186x; 1.0346x over previous
import jax
import jax.numpy as jnp
from jax import lax
from jax.experimental import pallas as pl
from jax.experimental.pallas import tpu as pltpu, tpu_sc as plsc

_NC, _NS = 2, 16
_NW = _NC * _NS
_B = 16384
_DIM = 64
_OUTW = 3 * _DIM
_BPW = _B // _NW
_STREAM = 128
_NCHUNK = _BPW // _STREAM


def _gather_body(ent_hbm, rel_hbm, h_hbm, r_hbm, t_hbm, out_hbm,
                 hidx, ridx, tidx, hbuf, rbuf, tbuf,
                 gsem0, gsem1, gsem2, gsem3, isem, wsem):
    wid = lax.axis_index("s") * _NC + lax.axis_index("c")
    base = wid * _BPW
    i0 = pltpu.async_copy(h_hbm.at[pl.ds(base, _BPW)], hidx, isem)
    i1 = pltpu.async_copy(r_hbm.at[pl.ds(base, _BPW)], ridx, isem)
    i2 = pltpu.async_copy(t_hbm.at[pl.ds(base, _BPW)], tidx, isem)
    i0.wait()
    i1.wait()
    i2.wait()

    gsems = (gsem0, gsem1, gsem2, gsem3)
    gathers = []
    for c in range(_NCHUNK):
        s = pl.ds(c * _STREAM, _STREAM)
        gathers.append((
            pltpu.async_copy(ent_hbm.at[hidx.at[s]], hbuf.at[s], gsems[c]),
            pltpu.async_copy(rel_hbm.at[ridx.at[s]], rbuf.at[s], gsems[c]),
            pltpu.async_copy(ent_hbm.at[tidx.at[s]], tbuf.at[s], gsems[c]),
        ))

    writes = []
    for c in range(_NCHUNK):
        s = pl.ds(c * _STREAM, _STREAM)
        rows = pl.ds(base + c * _STREAM, _STREAM)
        for cp in gathers[c]:
            cp.wait()
        writes.append(pltpu.async_copy(
            hbuf.at[s], out_hbm.at[rows, pl.ds(0, _DIM)], wsem))
        writes.append(pltpu.async_copy(
            rbuf.at[s], out_hbm.at[rows, pl.ds(_DIM, _DIM)], wsem))
        writes.append(pltpu.async_copy(
            tbuf.at[s], out_hbm.at[rows, pl.ds(2 * _DIM, _DIM)], wsem))
    for cp in writes:
        cp.wait()


def kernel(entity_table, relation_table, h_idx, r_idx, t_idx, y):
    mesh = plsc.VectorSubcoreMesh(core_axis_name="c", subcore_axis_name="s")
    out = pl.kernel(
        _gather_body,
        out_type=jax.ShapeDtypeStruct((_B, _OUTW), jnp.float32),
        mesh=mesh,
        compiler_params=pltpu.CompilerParams(use_tc_tiling_on_sc=False),
        scratch_types=[
            pltpu.VMEM((_BPW,), jnp.int32),
            pltpu.VMEM((_BPW,), jnp.int32),
            pltpu.VMEM((_BPW,), jnp.int32),
            pltpu.VMEM((_BPW, _DIM), jnp.float32),
            pltpu.VMEM((_BPW, _DIM), jnp.float32),
            pltpu.VMEM((_BPW, _DIM), jnp.float32),
            pltpu.SemaphoreType.DMA,
            pltpu.SemaphoreType.DMA,
            pltpu.SemaphoreType.DMA,
            pltpu.SemaphoreType.DMA,
            pltpu.SemaphoreType.DMA,
            pltpu.SemaphoreType.DMA,
        ],
    )(entity_table, relation_table,
      h_idx.astype(jnp.int32), r_idx.astype(jnp.int32), t_idx.astype(jnp.int32))
    return out.reshape(-1)
